# baseline (speedup 1.0000x reference)
import math

import jax
import jax.numpy as jnp
from jax.experimental import pallas as pl
from jax.experimental.pallas import tpu as pltpu

_B, _N, _C = 2, 4096, 1024
_SUB = 512
_SCALE = 1.0 / math.sqrt(_C)


def _attention(qkvb):
    q = qkvb[:, :_C]
    k = qkvb[:, _C:2 * _C]
    v = qkvb[:, 2 * _C:]
    s = jax.lax.dot_general(
        q, k, (((1,), (1,)), ((), ())), preferred_element_type=jnp.float32
    )
    e = jnp.exp(s)
    d = jnp.sum(e, axis=-1, keepdims=True)
    p = (e * (1.0 / d)).astype(jnp.bfloat16)
    os = jax.lax.dot_general(
        p, v, (((1,), (0,)), ((), ())), preferred_element_type=jnp.float32
    )
    return os.astype(jnp.bfloat16), d


def _sel(nrows, ncols, stride):
    r = jax.lax.broadcasted_iota(jnp.int32, (nrows, ncols), 0)
    c = jax.lax.broadcasted_iota(jnp.int32, (nrows, ncols), 1)
    return (c == stride * r).astype(jnp.bfloat16)


def _extract(sel, m):
    g = jax.lax.dot_general(
        sel, m, (((1,), (0,)), ((), ())), preferred_element_type=jnp.float32
    )
    return g.astype(jnp.bfloat16)


def _body(x_ref, w_ref, out_ref,
          qsb_ref, qsc_ref, osa_ref, da_ref, osb_ref, db_ref,
          osc_ref, dc_ref):
    b = pl.program_id(0)
    s = pl.program_id(1)

    @pl.when(b < _B)
    def _attn_step():
        xg = x_ref[0].astype(jnp.bfloat16)
        qkv = jnp.dot(
            xg, w_ref[...], preferred_element_type=jnp.float32
        ).astype(jnp.bfloat16)
        ev = _extract(_sel(_SUB // 2, _SUB, 2), qkv)
        qsb_ref[pl.ds((s % 2) * (_SUB // 2), _SUB // 2), :] = ev
        c8 = _extract(_sel(_SUB // 8, _SUB, 8), qkv)
        qsc_ref[pl.ds(s * (_SUB // 8), _SUB // 8), :] = c8

        osa, da = _attention(qkv)
        osa_ref[pl.ds(b * _N + s * _SUB, _SUB), :] = osa
        da_ref[pl.ds(b * _N + s * _SUB, _SUB), :] = da.astype(jnp.bfloat16)

        @pl.when(s % 2 == 1)
        def _():
            osb, db = _attention(qsb_ref[...])
            osb_ref[pl.ds(b * (_N // 2) + (s // 2) * _SUB, _SUB), :] = osb
            db_ref[pl.ds(b * (_N // 2) + (s // 2) * _SUB, _SUB), :] = (
                db.astype(jnp.bfloat16))

        @pl.when(s == 7)
        def _():
            osc, dc = _attention(qsc_ref[...])
            osc_ref[pl.ds(b * _SUB, _SUB), :] = osc
            dc_ref[pl.ds(b * _SUB, _SUB), :] = dc.astype(jnp.bfloat16)

    @pl.when(b >= 1)
    def _mix_step():
        m = b - 1
        osa = osa_ref[pl.ds(m * _N + s * _SUB, _SUB), :]
        da = da_ref[pl.ds(m * _N + s * _SUB, _SUB), :].astype(jnp.float32)
        osb = osb_ref[pl.ds(m * (_N // 2) + s * (_SUB // 2), _SUB // 2), :]
        db = db_ref[
            pl.ds(m * (_N // 2) + s * (_SUB // 2), _SUB // 2), :
        ].astype(jnp.float32)
        osc = osc_ref[pl.ds(m * _SUB + s * (_SUB // 8), _SUB // 8), :]
        dc = dc_ref[
            pl.ds(m * _SUB + s * (_SUB // 8), _SUB // 8), :
        ].astype(jnp.float32)
        i = jax.lax.broadcasted_iota(jnp.int32, (_SUB, 1), 0)
        w2 = jnp.where((i % 2) == 0, jnp.repeat(db, 2, axis=0), 0.0)
        w8 = jnp.where((i % 8) == 0, jnp.repeat(dc, 8, axis=0), 0.0)
        inv = 1.0 / (da + w2 + w8)
        aa = (da * inv).astype(jnp.bfloat16)
        ab = (w2 * inv).astype(jnp.bfloat16)
        ac = (w8 * inv).astype(jnp.bfloat16)
        blend = (aa * osa
                 + ab * jnp.repeat(osb, 2, axis=0)
                 + ac * jnp.repeat(osc, 8, axis=0))
        out_ref[0] = blend.astype(jnp.float32)


def _dilated_attention(x, wq, wk, wv, interpret=False):
    w = jnp.concatenate([wq * _SCALE, wk, wv], axis=1).astype(jnp.bfloat16)
    return pl.pallas_call(
        _body,
        grid=(_B + 1, 8),
        in_specs=[
            pl.BlockSpec(
                (1, _SUB, _C),
                lambda b, s: (jnp.minimum(b, _B - 1),
                              jnp.where(b >= _B, 7, s), 0),
            ),
            pl.BlockSpec((_C, 3 * _C), lambda b, s: (0, 0)),
        ],
        out_specs=pl.BlockSpec(
            (1, _SUB, _C),
            lambda b, s: (jnp.maximum(b - 1, 0),
                          jnp.where(b == 0, 0, s), 0),
        ),
        out_shape=jax.ShapeDtypeStruct((_B, _N, _C), jnp.float32),
        scratch_shapes=[
            pltpu.VMEM((_SUB, 3 * _C), jnp.bfloat16),
            pltpu.VMEM((_SUB, 3 * _C), jnp.bfloat16),
            pltpu.VMEM((_B * _N, _C), jnp.bfloat16),
            pltpu.VMEM((_B * _N, 1), jnp.bfloat16),
            pltpu.VMEM((_B * (_N // 2), _C), jnp.bfloat16),
            pltpu.VMEM((_B * (_N // 2), 1), jnp.bfloat16),
            pltpu.VMEM((_B * _SUB, _C), jnp.bfloat16),
            pltpu.VMEM((_B * _SUB, 1), jnp.bfloat16),
        ],
        interpret=interpret,
    )(x, w)


def kernel(x, Wq, Wk, Wv):
    return _dilated_attention(x, Wq, Wk, Wv)


# speedup vs baseline: 1.0662x; 1.0662x over previous
import math

import jax
import jax.numpy as jnp
from jax.experimental import pallas as pl
from jax.experimental.pallas import tpu as pltpu

_B, _N, _C = 2, 4096, 1024
_SUB = 512
_SCALE = 1.0 / math.sqrt(_C)


def _attention(qkvb):
    q = qkvb[:, :_C]
    k = qkvb[:, _C:2 * _C]
    v = qkvb[:, 2 * _C:]
    s = jax.lax.dot_general(
        q, k, (((1,), (1,)), ((), ())), preferred_element_type=jnp.float32
    )
    e = jnp.exp(s)
    d = jnp.sum(e, axis=-1, keepdims=True)
    p = (e * (1.0 / d)).astype(jnp.bfloat16)
    os = jax.lax.dot_general(
        p, v, (((1,), (0,)), ((), ())), preferred_element_type=jnp.float32
    )
    return os.astype(jnp.bfloat16), d


def _sel(nrows, ncols, stride):
    r = jax.lax.broadcasted_iota(jnp.int32, (nrows, ncols), 0)
    c = jax.lax.broadcasted_iota(jnp.int32, (nrows, ncols), 1)
    return (c == stride * r).astype(jnp.bfloat16)


def _extract(sel, m):
    g = jax.lax.dot_general(
        sel, m, (((1,), (0,)), ((), ())), preferred_element_type=jnp.float32
    )
    return g.astype(jnp.bfloat16)


def _body(x_ref, w_ref, out_ref,
          qsb_ref, qsc_ref, osa_ref, da_ref, osb_ref, db_ref,
          osc_ref, dc_ref):
    b = pl.program_id(0)
    s = pl.program_id(1)

    @pl.when(b < _B)
    def _attn_step():
        xg = x_ref[0].astype(jnp.bfloat16)
        qkv = jnp.dot(
            xg, w_ref[...], preferred_element_type=jnp.float32
        ).astype(jnp.bfloat16)
        ev = _extract(_sel(_SUB // 2, _SUB, 2), qkv)
        qsb_ref[pl.ds((s % 2) * (_SUB // 2), _SUB // 2), :] = ev
        c8 = _extract(_sel(_SUB // 8, _SUB, 8), qkv)
        qsc_ref[pl.ds(s * (_SUB // 8), _SUB // 8), :] = c8

        osa, da = _attention(qkv)
        osa_ref[pl.ds(b * _N + s * _SUB, _SUB), :] = osa
        da_ref[pl.ds(b * _N + s * _SUB, _SUB), :] = da

        @pl.when(s % 2 == 1)
        def _():
            osb, db = _attention(qsb_ref[...])
            osb_ref[pl.ds(b * (_N // 2) + (s // 2) * _SUB, _SUB), :] = osb
            db_ref[pl.ds(b * (_N // 2) + (s // 2) * _SUB, _SUB), :] = db

        @pl.when(s == 7)
        def _():
            osc, dc = _attention(qsc_ref[...])
            osc_ref[pl.ds(b * _SUB, _SUB), :] = osc
            dc_ref[pl.ds(b * _SUB, _SUB), :] = dc

    @pl.when(b >= 1)
    def _mix_step():
        m = b - 1
        osa = osa_ref[pl.ds(m * _N + s * _SUB, _SUB), :]
        da = da_ref[pl.ds(m * _N + s * _SUB, _SUB), :]
        osb = osb_ref[pl.ds(m * (_N // 2) + s * (_SUB // 2), _SUB // 2), :]
        db = db_ref[pl.ds(m * (_N // 2) + s * (_SUB // 2), _SUB // 2), :]
        osc = osc_ref[pl.ds(m * _SUB + s * (_SUB // 8), _SUB // 8), :]
        dc = dc_ref[pl.ds(m * _SUB + s * (_SUB // 8), _SUB // 8), :]
        i = jax.lax.broadcasted_iota(jnp.int32, (_SUB, 1), 0)
        w2 = jnp.where((i % 2) == 0, jnp.repeat(db, 2, axis=0), 0.0)
        w8 = jnp.where((i % 8) == 0, jnp.repeat(dc, 8, axis=0), 0.0)
        inv = 1.0 / (da + w2 + w8)
        out_ref[0] = ((da * inv) * osa.astype(jnp.float32)
                      + (w2 * inv) * jnp.repeat(osb, 2, axis=0).astype(jnp.float32)
                      + (w8 * inv) * jnp.repeat(osc, 8, axis=0).astype(jnp.float32))


def _dilated_attention(x, wq, wk, wv, interpret=False):
    w = jnp.concatenate([wq * _SCALE, wk, wv], axis=1).astype(jnp.bfloat16)
    return pl.pallas_call(
        _body,
        grid=(_B + 1, 8),
        in_specs=[
            pl.BlockSpec(
                (1, _SUB, _C),
                lambda b, s: (jnp.minimum(b, _B - 1),
                              jnp.where(b >= _B, 7, s), 0),
            ),
            pl.BlockSpec((_C, 3 * _C), lambda b, s: (0, 0)),
        ],
        out_specs=pl.BlockSpec(
            (1, _SUB, _C),
            lambda b, s: (jnp.maximum(b - 1, 0),
                          jnp.where(b == 0, 0, s), 0),
        ),
        out_shape=jax.ShapeDtypeStruct((_B, _N, _C), jnp.float32),
        scratch_shapes=[
            pltpu.VMEM((_SUB, 3 * _C), jnp.bfloat16),
            pltpu.VMEM((_SUB, 3 * _C), jnp.bfloat16),
            pltpu.VMEM((_B * _N, _C), jnp.bfloat16),
            pltpu.VMEM((_B * _N, 1), jnp.float32),
            pltpu.VMEM((_B * (_N // 2), _C), jnp.bfloat16),
            pltpu.VMEM((_B * (_N // 2), 1), jnp.float32),
            pltpu.VMEM((_B * _SUB, _C), jnp.bfloat16),
            pltpu.VMEM((_B * _SUB, 1), jnp.float32),
        ],
        interpret=interpret,
    )(x, w)


def kernel(x, Wq, Wk, Wv):
    return _dilated_attention(x, Wq, Wk, Wv)


# speedup vs baseline: 1.0880x; 1.0204x over previous
import math

import jax
import jax.numpy as jnp
from jax.experimental import pallas as pl
from jax.experimental.pallas import tpu as pltpu

_B, _N, _C = 2, 4096, 1024
_SUB = 512
_SCALE = 1.0 / math.sqrt(_C)


def _attention(qkvb):
    q = qkvb[:, :_C]
    k = qkvb[:, _C:2 * _C]
    v = qkvb[:, 2 * _C:]
    s = jax.lax.dot_general(
        q, k, (((1,), (1,)), ((), ())), preferred_element_type=jnp.float32
    )
    e = jnp.exp(s)
    d = jnp.sum(e, axis=-1, keepdims=True)
    p = (e * (1.0 / d)).astype(jnp.bfloat16)
    os = jax.lax.dot_general(
        p, v, (((1,), (0,)), ((), ())), preferred_element_type=jnp.float32
    )
    return os.astype(jnp.bfloat16), d


def _sel(nrows, ncols, stride):
    r = jax.lax.broadcasted_iota(jnp.int32, (nrows, ncols), 0)
    c = jax.lax.broadcasted_iota(jnp.int32, (nrows, ncols), 1)
    return (c == stride * r).astype(jnp.bfloat16)


def _extract(sel, m):
    g = jax.lax.dot_general(
        sel, m, (((1,), (0,)), ((), ())), preferred_element_type=jnp.float32
    )
    return g.astype(jnp.bfloat16)


def _body(x_ref, w_ref, out_ref,
          qsb_ref, qsc_ref, osa_ref, da_ref, osb_ref, db_ref,
          osc_ref, dc_ref):
    b = pl.program_id(0)
    s = pl.program_id(1)

    @pl.when(b < _B)
    def _attn_step():
        xg = x_ref[0].astype(jnp.bfloat16)
        qkv = jnp.dot(
            xg, w_ref[...], preferred_element_type=jnp.float32
        ).astype(jnp.bfloat16)
        ev = _extract(_sel(_SUB // 2, _SUB, 2), qkv)
        qsb_ref[pl.ds((s % 2) * (_SUB // 2), _SUB // 2), :] = ev
        c8 = _extract(_sel(_SUB // 8, _SUB, 8), qkv)
        qsc_ref[pl.ds(s * (_SUB // 8), _SUB // 8), :] = c8

        osa, da = _attention(qkv)
        osa_ref[pl.ds(b * _N + s * _SUB, _SUB), :] = osa
        da_ref[pl.ds(b * _N + s * _SUB, _SUB), :] = da

        @pl.when(s % 2 == 1)
        def _():
            osb, db = _attention(qsb_ref[...])
            osb_ref[pl.ds(b * (_N // 2) + (s // 2) * _SUB, _SUB), :] = osb
            db_ref[pl.ds(b * (_N // 2) + (s // 2) * _SUB, _SUB), :] = db

        @pl.when(s == 7)
        def _():
            osc, dc = _attention(qsc_ref[...])
            osc_ref[pl.ds(b * _SUB, _SUB), :] = osc
            dc_ref[pl.ds(b * _SUB, _SUB), :] = dc

    @pl.when(b >= 1)
    def _mix_step():
        m = b - 1
        osa = osa_ref[pl.ds(m * _N + s * _SUB, _SUB), :]
        da = da_ref[pl.ds(m * _N + s * _SUB, _SUB), :]
        osb = osb_ref[pl.ds(m * (_N // 2) + s * (_SUB // 2), _SUB // 2), :]
        db = db_ref[pl.ds(m * (_N // 2) + s * (_SUB // 2), _SUB // 2), :]
        osc = osc_ref[pl.ds(m * _SUB + s * (_SUB // 8), _SUB // 8), :]
        dc = dc_ref[pl.ds(m * _SUB + s * (_SUB // 8), _SUB // 8), :]
        i = jax.lax.broadcasted_iota(jnp.int32, (_SUB, 1), 0)
        w2 = jnp.where((i % 2) == 0, jnp.repeat(db, 2, axis=0), 0.0)
        w8 = jnp.where((i % 8) == 0, jnp.repeat(dc, 8, axis=0), 0.0)
        inv = 1.0 / (da + w2 + w8)
        out_ref[0] = ((da * inv) * osa.astype(jnp.float32)
                      + (w2 * inv) * jnp.repeat(osb.astype(jnp.float32), 2, axis=0)
                      + (w8 * inv) * jnp.repeat(osc.astype(jnp.float32), 8, axis=0))


def _dilated_attention(x, wq, wk, wv, interpret=False):
    w = jnp.concatenate([wq * _SCALE, wk, wv], axis=1).astype(jnp.bfloat16)
    return pl.pallas_call(
        _body,
        grid=(_B + 1, 8),
        in_specs=[
            pl.BlockSpec(
                (1, _SUB, _C),
                lambda b, s: (jnp.minimum(b, _B - 1),
                              jnp.where(b >= _B, 7, s), 0),
            ),
            pl.BlockSpec((_C, 3 * _C), lambda b, s: (0, 0)),
        ],
        out_specs=pl.BlockSpec(
            (1, _SUB, _C),
            lambda b, s: (jnp.maximum(b - 1, 0),
                          jnp.where(b == 0, 0, s), 0),
        ),
        out_shape=jax.ShapeDtypeStruct((_B, _N, _C), jnp.float32),
        scratch_shapes=[
            pltpu.VMEM((_SUB, 3 * _C), jnp.bfloat16),
            pltpu.VMEM((_SUB, 3 * _C), jnp.bfloat16),
            pltpu.VMEM((_B * _N, _C), jnp.bfloat16),
            pltpu.VMEM((_B * _N, 1), jnp.float32),
            pltpu.VMEM((_B * (_N // 2), _C), jnp.bfloat16),
            pltpu.VMEM((_B * (_N // 2), 1), jnp.float32),
            pltpu.VMEM((_B * _SUB, _C), jnp.bfloat16),
            pltpu.VMEM((_B * _SUB, 1), jnp.float32),
        ],
        interpret=interpret,
    )(x, w)


def kernel(x, Wq, Wk, Wv):
    return _dilated_attention(x, Wq, Wk, Wv)


# speedup vs baseline: 1.0891x; 1.0010x over previous
import math

import jax
import jax.numpy as jnp
from jax.experimental import pallas as pl
from jax.experimental.pallas import tpu as pltpu

_B, _N, _C = 2, 4096, 1024
_SUB = 512
_SCALE = 1.0 / math.sqrt(_C)


def _attention(qkvb):
    q = qkvb[:, :_C]
    k = qkvb[:, _C:2 * _C]
    v = qkvb[:, 2 * _C:]
    s1 = jax.lax.dot_general(
        q, k[:_SUB // 2], (((1,), (1,)), ((), ())),
        preferred_element_type=jnp.float32,
    )
    s2 = jax.lax.dot_general(
        q, k[_SUB // 2:], (((1,), (1,)), ((), ())),
        preferred_element_type=jnp.float32,
    )
    e1 = jnp.exp(s1)
    e2 = jnp.exp(s2)
    d = (jnp.sum(e1, axis=-1, keepdims=True)
         + jnp.sum(e2, axis=-1, keepdims=True))
    inv = 1.0 / d
    p = jnp.concatenate(
        [(e1 * inv).astype(jnp.bfloat16), (e2 * inv).astype(jnp.bfloat16)],
        axis=1,
    )
    os = jax.lax.dot_general(
        p, v, (((1,), (0,)), ((), ())), preferred_element_type=jnp.float32
    )
    return os.astype(jnp.bfloat16), d


def _sel(nrows, ncols, stride):
    r = jax.lax.broadcasted_iota(jnp.int32, (nrows, ncols), 0)
    c = jax.lax.broadcasted_iota(jnp.int32, (nrows, ncols), 1)
    return (c == stride * r).astype(jnp.bfloat16)


def _extract(sel, m):
    g = jax.lax.dot_general(
        sel, m, (((1,), (0,)), ((), ())), preferred_element_type=jnp.float32
    )
    return g.astype(jnp.bfloat16)


def _body(x_ref, w_ref, out_ref,
          qsb_ref, qsc_ref, osa_ref, da_ref, osb_ref, db_ref,
          osc_ref, dc_ref):
    b = pl.program_id(0)
    s = pl.program_id(1)

    @pl.when(b < _B)
    def _attn_step():
        xg = x_ref[0].astype(jnp.bfloat16)
        qkv = jnp.dot(
            xg, w_ref[...], preferred_element_type=jnp.float32
        ).astype(jnp.bfloat16)
        ev = _extract(_sel(_SUB // 2, _SUB, 2), qkv)
        qsb_ref[pl.ds((s % 2) * (_SUB // 2), _SUB // 2), :] = ev
        c8 = _extract(_sel(_SUB // 8, _SUB, 8), qkv)
        qsc_ref[pl.ds(s * (_SUB // 8), _SUB // 8), :] = c8

        osa, da = _attention(qkv)
        osa_ref[pl.ds(b * _N + s * _SUB, _SUB), :] = osa
        da_ref[pl.ds(b * _N + s * _SUB, _SUB), :] = da

        @pl.when(s % 2 == 1)
        def _():
            osb, db = _attention(qsb_ref[...])
            osb_ref[pl.ds(b * (_N // 2) + (s // 2) * _SUB, _SUB), :] = osb
            db_ref[pl.ds(b * (_N // 2) + (s // 2) * _SUB, _SUB), :] = db

        @pl.when(s == 7)
        def _():
            osc, dc = _attention(qsc_ref[...])
            osc_ref[pl.ds(b * _SUB, _SUB), :] = osc
            dc_ref[pl.ds(b * _SUB, _SUB), :] = dc

    @pl.when(b >= 1)
    def _mix_step():
        m = b - 1
        osa = osa_ref[pl.ds(m * _N + s * _SUB, _SUB), :]
        da = da_ref[pl.ds(m * _N + s * _SUB, _SUB), :]
        osb = osb_ref[pl.ds(m * (_N // 2) + s * (_SUB // 2), _SUB // 2), :]
        db = db_ref[pl.ds(m * (_N // 2) + s * (_SUB // 2), _SUB // 2), :]
        osc = osc_ref[pl.ds(m * _SUB + s * (_SUB // 8), _SUB // 8), :]
        dc = dc_ref[pl.ds(m * _SUB + s * (_SUB // 8), _SUB // 8), :]
        i = jax.lax.broadcasted_iota(jnp.int32, (_SUB, 1), 0)
        w2 = jnp.where((i % 2) == 0, jnp.repeat(db, 2, axis=0), 0.0)
        w8 = jnp.where((i % 8) == 0, jnp.repeat(dc, 8, axis=0), 0.0)
        inv = 1.0 / (da + w2 + w8)
        out_ref[0] = ((da * inv) * osa.astype(jnp.float32)
                      + (w2 * inv) * jnp.repeat(osb.astype(jnp.float32), 2, axis=0)
                      + (w8 * inv) * jnp.repeat(osc.astype(jnp.float32), 8, axis=0))


def _dilated_attention(x, wq, wk, wv, interpret=False):
    w = jnp.concatenate([wq * _SCALE, wk, wv], axis=1).astype(jnp.bfloat16)
    return pl.pallas_call(
        _body,
        grid=(_B + 1, 8),
        in_specs=[
            pl.BlockSpec(
                (1, _SUB, _C),
                lambda b, s: (jnp.minimum(b, _B - 1),
                              jnp.where(b >= _B, 7, s), 0),
            ),
            pl.BlockSpec((_C, 3 * _C), lambda b, s: (0, 0)),
        ],
        out_specs=pl.BlockSpec(
            (1, _SUB, _C),
            lambda b, s: (jnp.maximum(b - 1, 0),
                          jnp.where(b == 0, 0, s), 0),
        ),
        out_shape=jax.ShapeDtypeStruct((_B, _N, _C), jnp.float32),
        scratch_shapes=[
            pltpu.VMEM((_SUB, 3 * _C), jnp.bfloat16),
            pltpu.VMEM((_SUB, 3 * _C), jnp.bfloat16),
            pltpu.VMEM((_B * _N, _C), jnp.bfloat16),
            pltpu.VMEM((_B * _N, 1), jnp.float32),
            pltpu.VMEM((_B * (_N // 2), _C), jnp.bfloat16),
            pltpu.VMEM((_B * (_N // 2), 1), jnp.float32),
            pltpu.VMEM((_B * _SUB, _C), jnp.bfloat16),
            pltpu.VMEM((_B * _SUB, 1), jnp.float32),
        ],
        interpret=interpret,
    )(x, w)


def kernel(x, Wq, Wk, Wv):
    return _dilated_attention(x, Wq, Wk, Wv)
